# speedup vs baseline: 4.1223x; 4.1223x over previous
import functools

import jax
import jax.numpy as jnp
from jax import lax
from jax.experimental import pallas as pl
from jax.experimental.pallas import tpu as pltpu
from jax.experimental.pallas import tpu_sc as plsc

N = 10000
C = 128
E = 320000
NP = 10240
TRASH = 10016

NC = 2
NS = 16
NW = NC * NS
CH = 128
CPW = 79
EPW = CPW * CH
EP = NW * EPW
ZR = NP // NS

BR = 1024
GRID = NP // BR

_f32 = jnp.float32
_mesh = plsc.VectorSubcoreMesh(core_axis_name="c", subcore_axis_name="s")


def _make_scatter(with_deg: bool):
    out_type = [jax.ShapeDtypeStruct((NC, NP, C), _f32)]
    scratch = [
        pltpu.VMEM_SHARED((NP, C), _f32),
        pltpu.VMEM((CH,), jnp.int32),
        pltpu.VMEM((CH,), jnp.int32),
        pltpu.VMEM((CH, C), _f32),
        pltpu.SemaphoreType.DMA,
    ]
    if with_deg:
        out_type.append(jax.ShapeDtypeStruct((NC, NP), _f32))
        scratch += [
            pltpu.VMEM_SHARED((NP,), _f32),
            pltpu.VMEM((CH,), _f32),
        ]

    def body(z_hbm, src_hbm, dst_hbm, zrows_hbm, zvec_hbm, *rest):
        if with_deg:
            acc_out, deg_out, acc_sh, src_v, dst_v, rows_v, sem, deg_sh, ones_v = rest
        else:
            acc_out, acc_sh, src_v, dst_v, rows_v, sem = rest
        cid = lax.axis_index("c")
        sid = lax.axis_index("s")
        wid = cid * NS + sid

        pltpu.sync_copy(zrows_hbm, acc_sh.at[pl.ds(sid * ZR, ZR)])
        if with_deg:
            pltpu.sync_copy(zvec_hbm, deg_sh.at[pl.ds(sid * ZR, ZR)])
            for i in range(CH // 16):
                ones_v[pl.ds(i * 16, 16)] = jnp.full((16,), 1.0, _f32)
        plsc.subcore_barrier()

        base = wid * EPW

        def chunk(j, carry):
            off = base + j * CH
            pltpu.sync_copy(src_hbm.at[pl.ds(off, CH)], src_v)
            pltpu.sync_copy(dst_hbm.at[pl.ds(off, CH)], dst_v)
            pltpu.async_copy(z_hbm.at[src_v], rows_v, sem).wait()
            pltpu.sync_copy(rows_v, acc_sh.at[dst_v], add=True)
            if with_deg:
                pltpu.sync_copy(ones_v, deg_sh.at[dst_v], add=True)
            return carry

        lax.fori_loop(0, CPW, chunk, 0)
        plsc.subcore_barrier()

        pltpu.sync_copy(acc_sh.at[pl.ds(sid * ZR, ZR)],
                        acc_out.at[cid, pl.ds(sid * ZR, ZR)])
        if with_deg:
            pltpu.sync_copy(deg_sh.at[pl.ds(sid * ZR, ZR)],
                            deg_out.at[cid, pl.ds(sid * ZR, ZR)])

    return pl.kernel(body, out_type=out_type, mesh=_mesh,
                     scratch_types=scratch)


_scatter_deg = _make_scatter(True)
_scatter = _make_scatter(False)


def _dense_in_body(x_ref, wl_ref, wr_ref, b_ref, z_ref, r_ref):
    xb = x_ref[...]
    z_ref[...] = jnp.dot(xb, wl_ref[...], preferred_element_type=_f32)
    r_ref[...] = jnp.dot(xb, wr_ref[...], preferred_element_type=_f32) + b_ref[...]


_dense_in = pl.pallas_call(
    _dense_in_body,
    grid=(GRID,),
    in_specs=[
        pl.BlockSpec((BR, C), lambda i: (i, 0)),
        pl.BlockSpec((C, C), lambda i: (0, 0)),
        pl.BlockSpec((C, C), lambda i: (0, 0)),
        pl.BlockSpec((1, C), lambda i: (0, 0)),
    ],
    out_specs=[
        pl.BlockSpec((BR, C), lambda i: (i, 0)),
        pl.BlockSpec((BR, C), lambda i: (i, 0)),
    ],
    out_shape=[
        jax.ShapeDtypeStruct((NP, C), _f32),
        jax.ShapeDtypeStruct((NP, C), _f32),
    ],
)


def _combine(acc_ref, deg_ref, r_ref):
    d = jnp.maximum(deg_ref[0, :] + deg_ref[1, :], 1.0)
    agg = (acc_ref[0] + acc_ref[1]) / d[:, None]
    return agg + r_ref[...]


def _dense_mid_body(acc_ref, deg_ref, r_ref, wl_ref, wr_ref, b_ref,
                    z_ref, r2_ref):
    h = jnp.maximum(_combine(acc_ref, deg_ref, r_ref), 0.0)
    z_ref[...] = jnp.dot(h, wl_ref[...], preferred_element_type=_f32)
    r2_ref[...] = jnp.dot(h, wr_ref[...], preferred_element_type=_f32) + b_ref[...]


_dense_mid = pl.pallas_call(
    _dense_mid_body,
    grid=(GRID,),
    in_specs=[
        pl.BlockSpec((NC, BR, C), lambda i: (0, i, 0)),
        pl.BlockSpec((NC, BR), lambda i: (0, i)),
        pl.BlockSpec((BR, C), lambda i: (i, 0)),
        pl.BlockSpec((C, C), lambda i: (0, 0)),
        pl.BlockSpec((C, C), lambda i: (0, 0)),
        pl.BlockSpec((1, C), lambda i: (0, 0)),
    ],
    out_specs=[
        pl.BlockSpec((BR, C), lambda i: (i, 0)),
        pl.BlockSpec((BR, C), lambda i: (i, 0)),
    ],
    out_shape=[
        jax.ShapeDtypeStruct((NP, C), _f32),
        jax.ShapeDtypeStruct((NP, C), _f32),
    ],
)


def _dense_out_body(acc_ref, deg_ref, r_ref, o_ref):
    o = _combine(acc_ref, deg_ref, r_ref)
    m = jnp.max(o, axis=-1, keepdims=True)
    s = jnp.sum(jnp.exp(o - m), axis=-1, keepdims=True)
    o_ref[...] = (o - m) - jnp.log(s)


_dense_out = pl.pallas_call(
    _dense_out_body,
    grid=(GRID,),
    in_specs=[
        pl.BlockSpec((NC, BR, C), lambda i: (0, i, 0)),
        pl.BlockSpec((NC, BR), lambda i: (0, i)),
        pl.BlockSpec((BR, C), lambda i: (i, 0)),
    ],
    out_specs=pl.BlockSpec((BR, C), lambda i: (i, 0)),
    out_shape=jax.ShapeDtypeStruct((NP, C), _f32),
)


@jax.jit
def kernel(x, edge_index, W1_l, W1_r, b1, W2_l, W2_r, b2):
    src = edge_index[0].astype(jnp.int32)
    dst = edge_index[1].astype(jnp.int32)
    pad = EP - E
    src_p = jnp.concatenate([src, jnp.zeros((pad,), jnp.int32)])
    dst_p = jnp.concatenate([dst, jnp.full((pad,), TRASH, jnp.int32)])
    x_p = jnp.concatenate([x, jnp.zeros((NP - N, C), _f32)], axis=0)
    zrows = jnp.zeros((ZR, C), _f32)
    zvec = jnp.zeros((ZR,), _f32)
    b1r = b1.reshape(1, C)
    b2r = b2.reshape(1, C)

    z1, r1 = _dense_in(x_p, W1_l, W1_r, b1r)
    acc1, deg = _scatter_deg(z1, src_p, dst_p, zrows, zvec)
    z2, r2 = _dense_mid(acc1, deg, r1, W2_l, W2_r, b2r)
    (acc2,) = _scatter(z2, src_p, dst_p, zrows, zvec)
    out = _dense_out(acc2, deg, r2)
    return out[:N]


# speedup vs baseline: 4.4633x; 1.0827x over previous
import functools

import jax
import jax.numpy as jnp
from jax import lax
from jax.experimental import pallas as pl
from jax.experimental.pallas import tpu as pltpu
from jax.experimental.pallas import tpu_sc as plsc

N = 10000
C = 128
E = 320000
NP = 10240
TRASH = 10016

NC = 2
NS = 16
NW = NC * NS
CH = 128
CPW = 80
NB = 2
EPW = CPW * CH
EP = NW * EPW
ZR = NP // NS

BR = 1024
GRID = NP // BR

_f32 = jnp.float32
_mesh = plsc.VectorSubcoreMesh(core_axis_name="c", subcore_axis_name="s")


def _make_scatter(with_deg: bool):
    out_type = [jax.ShapeDtypeStruct((NC, NP, C), _f32)]
    scratch = (
        [pltpu.VMEM_SHARED((NP, C), _f32)]
        + [pltpu.VMEM((2, CH), jnp.int32)] * NB
        + [pltpu.VMEM((CH, C), _f32)] * NB
        + [pltpu.SemaphoreType.DMA] * NB
    )
    if with_deg:
        out_type.append(jax.ShapeDtypeStruct((NC, NP), _f32))
        scratch += [
            pltpu.VMEM_SHARED((NP,), _f32),
            pltpu.VMEM((CH,), _f32),
        ]

    def body(z_hbm, ei_hbm, zrows_hbm, zvec_hbm, *rest):
        if with_deg:
            acc_out, deg_out = rest[0], rest[1]
            k = 2
        else:
            (acc_out,) = rest[:1]
            k = 1
        idx = list(rest[k + 1:k + 1 + NB])
        rows = list(rest[k + 1 + NB:k + 1 + 2 * NB])
        sems = list(rest[k + 1 + 2 * NB:k + 1 + 3 * NB])
        acc_sh = rest[k]
        if with_deg:
            deg_sh, ones_v = rest[k + 1 + 3 * NB], rest[k + 2 + 3 * NB]
        cid = lax.axis_index("c")
        sid = lax.axis_index("s")
        wid = cid * NS + sid

        pltpu.sync_copy(zrows_hbm, acc_sh.at[pl.ds(sid * ZR, ZR)])
        if with_deg:
            pltpu.sync_copy(zvec_hbm, deg_sh.at[pl.ds(sid * ZR, ZR)])
            for i in range(CH // 16):
                ones_v[pl.ds(i * 16, 16)] = jnp.full((16,), 1.0, _f32)
        plsc.subcore_barrier()

        base = wid * CPW

        def load_idx(j, b):
            pltpu.sync_copy(ei_hbm.at[base + j], idx[b])

        def gather(b):
            pltpu.async_copy(z_hbm.at[idx[b].at[0]], rows[b], sems[b])

        def gather_wait(b):
            pltpu.make_async_copy(z_hbm.at[idx[b].at[0]], rows[b], sems[b]).wait()

        for b in range(NB):
            load_idx(b, b)
            gather(b)

        def group(g, carry):
            for b in range(NB):
                j = g * NB + b
                gather_wait(b)
                pltpu.sync_copy(rows[b], acc_sh.at[idx[b].at[1]], add=True)
                if with_deg:
                    pltpu.sync_copy(ones_v, deg_sh.at[idx[b].at[1]], add=True)
                load_idx(jnp.minimum(j + NB, CPW - 1), b)
                gather(b)
            return carry

        lax.fori_loop(0, CPW // NB, group, 0)
        for b in range(NB):
            gather_wait(b)
        plsc.subcore_barrier()

        pltpu.sync_copy(acc_sh.at[pl.ds(sid * ZR, ZR)],
                        acc_out.at[cid, pl.ds(sid * ZR, ZR)])
        if with_deg:
            pltpu.sync_copy(deg_sh.at[pl.ds(sid * ZR, ZR)],
                            deg_out.at[cid, pl.ds(sid * ZR, ZR)])

    return pl.kernel(body, out_type=out_type, mesh=_mesh,
                     scratch_types=scratch)


_scatter_deg = _make_scatter(True)
_scatter = _make_scatter(False)


def _dense_in_body(x_ref, wl_ref, wr_ref, b_ref, z_ref, r_ref):
    xb = x_ref[...]
    z_ref[...] = jnp.dot(xb, wl_ref[...], preferred_element_type=_f32)
    r_ref[...] = jnp.dot(xb, wr_ref[...], preferred_element_type=_f32) + b_ref[...]


_dense_in = pl.pallas_call(
    _dense_in_body,
    grid=(GRID,),
    in_specs=[
        pl.BlockSpec((BR, C), lambda i: (i, 0)),
        pl.BlockSpec((C, C), lambda i: (0, 0)),
        pl.BlockSpec((C, C), lambda i: (0, 0)),
        pl.BlockSpec((1, C), lambda i: (0, 0)),
    ],
    out_specs=[
        pl.BlockSpec((BR, C), lambda i: (i, 0)),
        pl.BlockSpec((BR, C), lambda i: (i, 0)),
    ],
    out_shape=[
        jax.ShapeDtypeStruct((NP, C), _f32),
        jax.ShapeDtypeStruct((NP, C), _f32),
    ],
)


def _combine(acc_ref, deg_ref, r_ref):
    d = jnp.maximum(deg_ref[0, :] + deg_ref[1, :], 1.0)
    agg = (acc_ref[0] + acc_ref[1]) / d[:, None]
    return agg + r_ref[...]


def _dense_mid_body(acc_ref, deg_ref, r_ref, wl_ref, wr_ref, b_ref,
                    z_ref, r2_ref):
    h = jnp.maximum(_combine(acc_ref, deg_ref, r_ref), 0.0)
    z_ref[...] = jnp.dot(h, wl_ref[...], preferred_element_type=_f32)
    r2_ref[...] = jnp.dot(h, wr_ref[...], preferred_element_type=_f32) + b_ref[...]


_dense_mid = pl.pallas_call(
    _dense_mid_body,
    grid=(GRID,),
    in_specs=[
        pl.BlockSpec((NC, BR, C), lambda i: (0, i, 0)),
        pl.BlockSpec((NC, BR), lambda i: (0, i)),
        pl.BlockSpec((BR, C), lambda i: (i, 0)),
        pl.BlockSpec((C, C), lambda i: (0, 0)),
        pl.BlockSpec((C, C), lambda i: (0, 0)),
        pl.BlockSpec((1, C), lambda i: (0, 0)),
    ],
    out_specs=[
        pl.BlockSpec((BR, C), lambda i: (i, 0)),
        pl.BlockSpec((BR, C), lambda i: (i, 0)),
    ],
    out_shape=[
        jax.ShapeDtypeStruct((NP, C), _f32),
        jax.ShapeDtypeStruct((NP, C), _f32),
    ],
)


def _dense_out_body(acc_ref, deg_ref, r_ref, o_ref):
    o = _combine(acc_ref, deg_ref, r_ref)
    m = jnp.max(o, axis=-1, keepdims=True)
    s = jnp.sum(jnp.exp(o - m), axis=-1, keepdims=True)
    o_ref[...] = (o - m) - jnp.log(s)


_dense_out = pl.pallas_call(
    _dense_out_body,
    grid=(GRID,),
    in_specs=[
        pl.BlockSpec((NC, BR, C), lambda i: (0, i, 0)),
        pl.BlockSpec((NC, BR), lambda i: (0, i)),
        pl.BlockSpec((BR, C), lambda i: (i, 0)),
    ],
    out_specs=pl.BlockSpec((BR, C), lambda i: (i, 0)),
    out_shape=jax.ShapeDtypeStruct((NP, C), _f32),
)


@jax.jit
def kernel(x, edge_index, W1_l, W1_r, b1, W2_l, W2_r, b2):
    src = edge_index[0].astype(jnp.int32)
    dst = edge_index[1].astype(jnp.int32)
    pad = EP - E
    src_p = jnp.concatenate([src, jnp.zeros((pad,), jnp.int32)]).reshape(NW * CPW, 1, CH)
    dst_p = jnp.concatenate([dst, jnp.full((pad,), TRASH, jnp.int32)]).reshape(NW * CPW, 1, CH)
    ei_p = jnp.concatenate([src_p, dst_p], axis=1)
    x_p = jnp.concatenate([x, jnp.zeros((NP - N, C), _f32)], axis=0)
    zrows = jnp.zeros((ZR, C), _f32)
    zvec = jnp.zeros((ZR,), _f32)
    b1r = b1.reshape(1, C)
    b2r = b2.reshape(1, C)

    z1, r1 = _dense_in(x_p, W1_l, W1_r, b1r)
    acc1, deg = _scatter_deg(z1, ei_p, zrows, zvec)
    z2, r2 = _dense_mid(acc1, deg, r1, W2_l, W2_r, b2r)
    (acc2,) = _scatter(z2, ei_p, zrows, zvec)
    out = _dense_out(acc2, deg, r2)
    return out[:N]


# speedup vs baseline: 4.4774x; 1.0032x over previous
import functools

import jax
import jax.numpy as jnp
from jax import lax
from jax.experimental import pallas as pl
from jax.experimental.pallas import tpu as pltpu
from jax.experimental.pallas import tpu_sc as plsc

N = 10000
C = 128
E = 320000
NP = 10240
TRASH = 10016

NC = 2
NS = 16
NW = NC * NS
CH = 128
CPW = 80
NB = 2
CPW0 = 118
CPW1 = 42
EP = NW * CPW * CH
ZR = NP // NS

BR = 1024
GRID = NP // BR

_f32 = jnp.float32
_mesh = plsc.VectorSubcoreMesh(core_axis_name="c", subcore_axis_name="s")


def _make_scatter(with_deg: bool):
    out_type = [jax.ShapeDtypeStruct((NC, NP, C), _f32)]
    scratch = (
        [pltpu.VMEM_SHARED((NP, C), _f32)]
        + [pltpu.VMEM((2, CH), jnp.int32)] * NB
        + [pltpu.VMEM((CH, C), _f32)] * NB
        + [pltpu.SemaphoreType.DMA] * NB
    )
    if with_deg:
        out_type.append(jax.ShapeDtypeStruct((NC, NP), _f32))
        scratch += [
            pltpu.VMEM_SHARED((NP,), _f32),
            pltpu.VMEM((CH,), _f32),
        ]

    def body(z_hbm, ei_hbm, zrows_hbm, zvec_hbm, *rest):
        if with_deg:
            acc_out, deg_out = rest[0], rest[1]
            k = 2
        else:
            (acc_out,) = rest[:1]
            k = 1
        idx = list(rest[k + 1:k + 1 + NB])
        rows = list(rest[k + 1 + NB:k + 1 + 2 * NB])
        sems = list(rest[k + 1 + 2 * NB:k + 1 + 3 * NB])
        acc_sh = rest[k]
        if with_deg:
            deg_sh, ones_v = rest[k + 1 + 3 * NB], rest[k + 2 + 3 * NB]
        cid = lax.axis_index("c")
        sid = lax.axis_index("s")
        wid = cid * NS + sid

        pltpu.sync_copy(zrows_hbm, acc_sh.at[pl.ds(sid * ZR, ZR)])
        if with_deg:
            pltpu.sync_copy(zvec_hbm, deg_sh.at[pl.ds(sid * ZR, ZR)])
            for i in range(CH // 16):
                ones_v[pl.ds(i * 16, 16)] = jnp.full((16,), 1.0, _f32)
        plsc.subcore_barrier()

        base = jnp.where(cid == 0, sid * CPW0, NS * CPW0 + sid * CPW1)
        cpw = jnp.where(cid == 0, CPW0, CPW1)

        def load_idx(j, b):
            pltpu.sync_copy(ei_hbm.at[base + j], idx[b])

        def gather(b):
            pltpu.async_copy(z_hbm.at[idx[b].at[0]], rows[b], sems[b])

        def gather_wait(b):
            pltpu.make_async_copy(z_hbm.at[idx[b].at[0]], rows[b], sems[b]).wait()

        for b in range(NB):
            load_idx(jnp.minimum(b, cpw - 1), b)
            gather(b)

        def group(g, carry):
            for b in range(NB):
                j = g * NB + b
                gather_wait(b)
                pltpu.sync_copy(rows[b], acc_sh.at[idx[b].at[1]], add=True)
                if with_deg:
                    pltpu.sync_copy(ones_v, deg_sh.at[idx[b].at[1]], add=True)
                load_idx(jnp.minimum(j + NB, cpw - 1), b)
                gather(b)
            return carry

        lax.fori_loop(0, cpw // NB, group, 0)
        for b in range(NB):
            gather_wait(b)
        plsc.subcore_barrier()

        pltpu.sync_copy(acc_sh.at[pl.ds(sid * ZR, ZR)],
                        acc_out.at[cid, pl.ds(sid * ZR, ZR)])
        if with_deg:
            pltpu.sync_copy(deg_sh.at[pl.ds(sid * ZR, ZR)],
                            deg_out.at[cid, pl.ds(sid * ZR, ZR)])

    return pl.kernel(body, out_type=out_type, mesh=_mesh,
                     scratch_types=scratch)


_scatter_deg = _make_scatter(True)
_scatter = _make_scatter(False)


def _dense_in_body(x_ref, wl_ref, wr_ref, b_ref, z_ref, r_ref):
    xb = x_ref[...]
    z_ref[...] = jnp.dot(xb, wl_ref[...], preferred_element_type=_f32)
    r_ref[...] = jnp.dot(xb, wr_ref[...], preferred_element_type=_f32) + b_ref[...]


_dense_in = pl.pallas_call(
    _dense_in_body,
    grid=(GRID,),
    in_specs=[
        pl.BlockSpec((BR, C), lambda i: (i, 0)),
        pl.BlockSpec((C, C), lambda i: (0, 0)),
        pl.BlockSpec((C, C), lambda i: (0, 0)),
        pl.BlockSpec((1, C), lambda i: (0, 0)),
    ],
    out_specs=[
        pl.BlockSpec((BR, C), lambda i: (i, 0)),
        pl.BlockSpec((BR, C), lambda i: (i, 0)),
    ],
    out_shape=[
        jax.ShapeDtypeStruct((NP, C), _f32),
        jax.ShapeDtypeStruct((NP, C), _f32),
    ],
)


def _combine(acc_ref, deg_ref, r_ref):
    d = jnp.maximum(deg_ref[0, :] + deg_ref[1, :], 1.0)
    agg = (acc_ref[0] + acc_ref[1]) / d[:, None]
    return agg + r_ref[...]


def _dense_mid_body(acc_ref, deg_ref, r_ref, wl_ref, wr_ref, b_ref,
                    z_ref, r2_ref):
    h = jnp.maximum(_combine(acc_ref, deg_ref, r_ref), 0.0)
    z_ref[...] = jnp.dot(h, wl_ref[...], preferred_element_type=_f32)
    r2_ref[...] = jnp.dot(h, wr_ref[...], preferred_element_type=_f32) + b_ref[...]


_dense_mid = pl.pallas_call(
    _dense_mid_body,
    grid=(GRID,),
    in_specs=[
        pl.BlockSpec((NC, BR, C), lambda i: (0, i, 0)),
        pl.BlockSpec((NC, BR), lambda i: (0, i)),
        pl.BlockSpec((BR, C), lambda i: (i, 0)),
        pl.BlockSpec((C, C), lambda i: (0, 0)),
        pl.BlockSpec((C, C), lambda i: (0, 0)),
        pl.BlockSpec((1, C), lambda i: (0, 0)),
    ],
    out_specs=[
        pl.BlockSpec((BR, C), lambda i: (i, 0)),
        pl.BlockSpec((BR, C), lambda i: (i, 0)),
    ],
    out_shape=[
        jax.ShapeDtypeStruct((NP, C), _f32),
        jax.ShapeDtypeStruct((NP, C), _f32),
    ],
)


def _dense_out_body(acc_ref, deg_ref, r_ref, o_ref):
    o = _combine(acc_ref, deg_ref, r_ref)
    m = jnp.max(o, axis=-1, keepdims=True)
    s = jnp.sum(jnp.exp(o - m), axis=-1, keepdims=True)
    o_ref[...] = (o - m) - jnp.log(s)


_dense_out = pl.pallas_call(
    _dense_out_body,
    grid=(GRID,),
    in_specs=[
        pl.BlockSpec((NC, BR, C), lambda i: (0, i, 0)),
        pl.BlockSpec((NC, BR), lambda i: (0, i)),
        pl.BlockSpec((BR, C), lambda i: (i, 0)),
    ],
    out_specs=pl.BlockSpec((BR, C), lambda i: (i, 0)),
    out_shape=jax.ShapeDtypeStruct((NP, C), _f32),
)


@jax.jit
def kernel(x, edge_index, W1_l, W1_r, b1, W2_l, W2_r, b2):
    src = edge_index[0].astype(jnp.int32)
    dst = edge_index[1].astype(jnp.int32)
    pad = EP - E
    src_p = jnp.concatenate([src, jnp.zeros((pad,), jnp.int32)]).reshape(NW * CPW, 1, CH)
    dst_p = jnp.concatenate([dst, jnp.full((pad,), TRASH, jnp.int32)]).reshape(NW * CPW, 1, CH)
    ei_p = jnp.concatenate([src_p, dst_p], axis=1)
    x_p = jnp.concatenate([x, jnp.zeros((NP - N, C), _f32)], axis=0)
    zrows = jnp.zeros((ZR, C), _f32)
    zvec = jnp.zeros((ZR,), _f32)
    b1r = b1.reshape(1, C)
    b2r = b2.reshape(1, C)

    z1, r1 = _dense_in(x_p, W1_l, W1_r, b1r)
    acc1, deg = _scatter_deg(z1, ei_p, zrows, zvec)
    z2, r2 = _dense_mid(acc1, deg, r1, W2_l, W2_r, b2r)
    (acc2,) = _scatter(z2, ei_p, zrows, zvec)
    out = _dense_out(acc2, deg, r2)
    return out[:N]


# speedup vs baseline: 4.5929x; 1.0258x over previous
import functools

import jax
import jax.numpy as jnp
from jax import lax
from jax.experimental import pallas as pl
from jax.experimental.pallas import tpu as pltpu
from jax.experimental.pallas import tpu_sc as plsc

N = 10000
C = 128
E = 320000
NP = 10240
TRASH = 10016

NC = 2
NS = 16
NW = NC * NS
CH = 128
CPW = 80
NB = 2
HALF = CPW // 2
EP = NW * CPW * CH
ZR = NP // NS

BR = 1024
GRID = NP // BR

_f32 = jnp.float32
_mesh = plsc.VectorSubcoreMesh(core_axis_name="c", subcore_axis_name="s")


def _make_scatter(with_deg: bool):
    out_type = [jax.ShapeDtypeStruct((NC, NP, C), _f32)]
    scratch = (
        [pltpu.VMEM_SHARED((NP, C), _f32)]
        + [pltpu.VMEM((HALF, 2, CH), jnp.int32)]
        + [pltpu.VMEM((CH, C), _f32)] * NB
        + [pltpu.SemaphoreType.DMA] * NB
    )
    if with_deg:
        out_type.append(jax.ShapeDtypeStruct((NC, NP), _f32))
        scratch += [
            pltpu.VMEM_SHARED((NP,), _f32),
            pltpu.VMEM((CH,), _f32),
        ]

    def body(z_hbm, ei_hbm, zrows_hbm, zvec_hbm, *rest):
        if with_deg:
            acc_out, deg_out = rest[0], rest[1]
            k = 2
        else:
            (acc_out,) = rest[:1]
            k = 1
        slab = rest[k + 1]
        rows = list(rest[k + 2:k + 2 + NB])
        sems = list(rest[k + 2 + NB:k + 2 + 2 * NB])
        acc_sh = rest[k]
        if with_deg:
            deg_sh, ones_v = rest[k + 2 + 2 * NB], rest[k + 3 + 2 * NB]
        cid = lax.axis_index("c")
        sid = lax.axis_index("s")
        wid = cid * NS + sid

        pltpu.sync_copy(zrows_hbm, acc_sh.at[pl.ds(sid * ZR, ZR)])
        if with_deg:
            pltpu.sync_copy(zvec_hbm, deg_sh.at[pl.ds(sid * ZR, ZR)])
            for i in range(CH // 16):
                ones_v[pl.ds(i * 16, 16)] = jnp.full((16,), 1.0, _f32)
        plsc.subcore_barrier()

        base = wid * CPW

        def gather(j, b):
            pltpu.async_copy(z_hbm.at[slab.at[j, 0]], rows[b], sems[b])

        def gather_wait(b):
            pltpu.make_async_copy(z_hbm.at[slab.at[0, 0]], rows[b], sems[b]).wait()

        def consume(j, b):
            gather_wait(b)
            pltpu.sync_copy(rows[b], acc_sh.at[slab.at[j, 1]], add=True)
            if with_deg:
                pltpu.sync_copy(ones_v, deg_sh.at[slab.at[j, 1]], add=True)

        for h in range(2):
            pltpu.sync_copy(ei_hbm.at[pl.ds(base + h * HALF, HALF)], slab)
            for b in range(NB):
                gather(b, b)

            def group(g, carry):
                for b in range(NB):
                    j = g * NB + b
                    consume(j, b)
                    gather(j + NB, b)
                return carry

            lax.fori_loop(0, (HALF - NB) // NB, group, 0)
            for b in range(NB):
                consume(HALF - NB + b, b)
        plsc.subcore_barrier()

        pltpu.sync_copy(acc_sh.at[pl.ds(sid * ZR, ZR)],
                        acc_out.at[cid, pl.ds(sid * ZR, ZR)])
        if with_deg:
            pltpu.sync_copy(deg_sh.at[pl.ds(sid * ZR, ZR)],
                            deg_out.at[cid, pl.ds(sid * ZR, ZR)])

    return pl.kernel(body, out_type=out_type, mesh=_mesh,
                     scratch_types=scratch)


_scatter_deg = _make_scatter(True)
_scatter = _make_scatter(False)


def _dense_in_body(x_ref, wl_ref, wr_ref, b_ref, z_ref, r_ref):
    xb = x_ref[...]
    z_ref[...] = jnp.dot(xb, wl_ref[...], preferred_element_type=_f32)
    r_ref[...] = jnp.dot(xb, wr_ref[...], preferred_element_type=_f32) + b_ref[...]


_dense_in = pl.pallas_call(
    _dense_in_body,
    grid=(GRID,),
    in_specs=[
        pl.BlockSpec((BR, C), lambda i: (i, 0)),
        pl.BlockSpec((C, C), lambda i: (0, 0)),
        pl.BlockSpec((C, C), lambda i: (0, 0)),
        pl.BlockSpec((1, C), lambda i: (0, 0)),
    ],
    out_specs=[
        pl.BlockSpec((BR, C), lambda i: (i, 0)),
        pl.BlockSpec((BR, C), lambda i: (i, 0)),
    ],
    out_shape=[
        jax.ShapeDtypeStruct((NP, C), _f32),
        jax.ShapeDtypeStruct((NP, C), _f32),
    ],
)


def _combine(acc_ref, deg_ref, r_ref):
    d = jnp.maximum(deg_ref[0, :] + deg_ref[1, :], 1.0)
    agg = (acc_ref[0] + acc_ref[1]) / d[:, None]
    return agg + r_ref[...]


def _dense_mid_body(acc_ref, deg_ref, r_ref, wl_ref, wr_ref, b_ref,
                    z_ref, r2_ref):
    h = jnp.maximum(_combine(acc_ref, deg_ref, r_ref), 0.0)
    z_ref[...] = jnp.dot(h, wl_ref[...], preferred_element_type=_f32)
    r2_ref[...] = jnp.dot(h, wr_ref[...], preferred_element_type=_f32) + b_ref[...]


_dense_mid = pl.pallas_call(
    _dense_mid_body,
    grid=(GRID,),
    in_specs=[
        pl.BlockSpec((NC, BR, C), lambda i: (0, i, 0)),
        pl.BlockSpec((NC, BR), lambda i: (0, i)),
        pl.BlockSpec((BR, C), lambda i: (i, 0)),
        pl.BlockSpec((C, C), lambda i: (0, 0)),
        pl.BlockSpec((C, C), lambda i: (0, 0)),
        pl.BlockSpec((1, C), lambda i: (0, 0)),
    ],
    out_specs=[
        pl.BlockSpec((BR, C), lambda i: (i, 0)),
        pl.BlockSpec((BR, C), lambda i: (i, 0)),
    ],
    out_shape=[
        jax.ShapeDtypeStruct((NP, C), _f32),
        jax.ShapeDtypeStruct((NP, C), _f32),
    ],
)


def _dense_out_body(acc_ref, deg_ref, r_ref, o_ref):
    o = _combine(acc_ref, deg_ref, r_ref)
    m = jnp.max(o, axis=-1, keepdims=True)
    s = jnp.sum(jnp.exp(o - m), axis=-1, keepdims=True)
    o_ref[...] = (o - m) - jnp.log(s)


_dense_out = pl.pallas_call(
    _dense_out_body,
    grid=(GRID,),
    in_specs=[
        pl.BlockSpec((NC, BR, C), lambda i: (0, i, 0)),
        pl.BlockSpec((NC, BR), lambda i: (0, i)),
        pl.BlockSpec((BR, C), lambda i: (i, 0)),
    ],
    out_specs=pl.BlockSpec((BR, C), lambda i: (i, 0)),
    out_shape=jax.ShapeDtypeStruct((NP, C), _f32),
)


@jax.jit
def kernel(x, edge_index, W1_l, W1_r, b1, W2_l, W2_r, b2):
    src = edge_index[0].astype(jnp.int32)
    dst = edge_index[1].astype(jnp.int32)
    pad = EP - E
    src_p = jnp.concatenate([src, jnp.zeros((pad,), jnp.int32)]).reshape(NW * CPW, 1, CH)
    dst_p = jnp.concatenate([dst, jnp.full((pad,), TRASH, jnp.int32)]).reshape(NW * CPW, 1, CH)
    ei_p = jnp.concatenate([src_p, dst_p], axis=1)
    x_p = jnp.concatenate([x, jnp.zeros((NP - N, C), _f32)], axis=0)
    zrows = jnp.zeros((ZR, C), _f32)
    zvec = jnp.zeros((ZR,), _f32)
    b1r = b1.reshape(1, C)
    b2r = b2.reshape(1, C)

    z1, r1 = _dense_in(x_p, W1_l, W1_r, b1r)
    acc1, deg = _scatter_deg(z1, ei_p, zrows, zvec)
    z2, r2 = _dense_mid(acc1, deg, r1, W2_l, W2_r, b2r)
    (acc2,) = _scatter(z2, ei_p, zrows, zvec)
    out = _dense_out(acc2, deg, r2)
    return out[:N]


# speedup vs baseline: 8.8385x; 1.9244x over previous
import functools

import jax
import jax.numpy as jnp
from jax import lax
from jax.experimental import pallas as pl
from jax.experimental.pallas import tpu as pltpu
from jax.experimental.pallas import tpu_sc as plsc

N = 10000
C = 128
E = 320000
NP = 10240
TRASH = 10016

NC = 2
NS = 16
NW = NC * NS
CH = 128
CPW = 160
NB = 2
HALF = 40
CHH = C // 2
EP = NS * CPW * CH
ZR = NP // NS

BR = 1024
GRID = NP // BR

_f32 = jnp.float32
_mesh = plsc.VectorSubcoreMesh(core_axis_name="c", subcore_axis_name="s")


def _make_scatter(with_deg: bool):
    out_type = [jax.ShapeDtypeStruct((NC, NP, CHH), _f32)]
    scratch = (
        [pltpu.VMEM_SHARED((NP, CHH), _f32)]
        + [pltpu.VMEM_SHARED((NP, CHH), _f32)]
        + [pltpu.VMEM((HALF, 2, CH), jnp.int32)]
        + [pltpu.VMEM((CH, CHH), _f32)] * NB
        + [pltpu.SemaphoreType.DMA] * NB
    )
    if with_deg:
        out_type.append(jax.ShapeDtypeStruct((NP,), _f32))
        scratch += [
            pltpu.VMEM_SHARED((NP,), _f32),
            pltpu.VMEM((CH,), _f32),
        ]

    def body(z0_hbm, z1_hbm, ei_hbm, zrows_hbm, zvec_hbm, *rest):
        if with_deg:
            acc_out, deg_out = rest[0], rest[1]
            k = 2
        else:
            (acc_out,) = rest[:1]
            k = 1
        acc_sh, z_sh, slab = rest[k], rest[k + 1], rest[k + 2]
        rows = list(rest[k + 3:k + 3 + NB])
        sems = list(rest[k + 3 + NB:k + 3 + 2 * NB])
        if with_deg:
            deg_sh, ones_v = rest[k + 3 + 2 * NB], rest[k + 4 + 2 * NB]
        cid = lax.axis_index("c")
        sid = lax.axis_index("s")

        pltpu.sync_copy(zrows_hbm, acc_sh.at[pl.ds(sid * ZR, ZR)])

        @pl.when(cid == 0)
        def _():
            pltpu.sync_copy(z0_hbm.at[pl.ds(sid * ZR, ZR)],
                            z_sh.at[pl.ds(sid * ZR, ZR)])

        @pl.when(cid == 1)
        def _():
            pltpu.sync_copy(z1_hbm.at[pl.ds(sid * ZR, ZR)],
                            z_sh.at[pl.ds(sid * ZR, ZR)])

        if with_deg:
            @pl.when(cid == 0)
            def _():
                pltpu.sync_copy(zvec_hbm, deg_sh.at[pl.ds(sid * ZR, ZR)])
            for i in range(CH // 16):
                ones_v[pl.ds(i * 16, 16)] = jnp.full((16,), 1.0, _f32)
        plsc.subcore_barrier()

        base = sid * CPW

        def gather(j, b):
            pltpu.async_copy(z_sh.at[slab.at[j, 0]], rows[b], sems[b])

        def gather_wait(b):
            pltpu.make_async_copy(z_sh.at[slab.at[0, 0]], rows[b], sems[b]).wait()

        def consume(j, b):
            gather_wait(b)
            pltpu.sync_copy(rows[b], acc_sh.at[slab.at[j, 1]], add=True)
            if with_deg:
                @pl.when(cid == 0)
                def _():
                    pltpu.sync_copy(ones_v, deg_sh.at[slab.at[j, 1]], add=True)

        for h in range(CPW // HALF):
            pltpu.sync_copy(ei_hbm.at[pl.ds(base + h * HALF, HALF)], slab)
            for b in range(NB):
                gather(b, b)

            def group(g, carry):
                for b in range(NB):
                    j = g * NB + b
                    consume(j, b)
                    gather(j + NB, b)
                return carry

            lax.fori_loop(0, (HALF - NB) // NB, group, 0)
            for b in range(NB):
                consume(HALF - NB + b, b)
        plsc.subcore_barrier()

        pltpu.sync_copy(acc_sh.at[pl.ds(sid * ZR, ZR)],
                        acc_out.at[cid, pl.ds(sid * ZR, ZR)])
        if with_deg:
            @pl.when(cid == 0)
            def _():
                pltpu.sync_copy(deg_sh.at[pl.ds(sid * ZR, ZR)],
                                deg_out.at[pl.ds(sid * ZR, ZR)])

    return pl.kernel(body, out_type=out_type, mesh=_mesh,
                     scratch_types=scratch)


_scatter_deg = _make_scatter(True)
_scatter = _make_scatter(False)


_zspec = [
    pl.BlockSpec((BR, CHH), lambda i: (i, 0)),
    pl.BlockSpec((BR, CHH), lambda i: (i, 0)),
]
_zshape = [
    jax.ShapeDtypeStruct((NP, CHH), _f32),
    jax.ShapeDtypeStruct((NP, CHH), _f32),
]


def _dense_in_body(x_ref, wl_ref, wr_ref, b_ref, z0_ref, z1_ref, r_ref):
    xb = x_ref[...]
    z = jnp.dot(xb, wl_ref[...], preferred_element_type=_f32)
    z0_ref[...] = z[:, :CHH]
    z1_ref[...] = z[:, CHH:]
    r_ref[...] = jnp.dot(xb, wr_ref[...], preferred_element_type=_f32) + b_ref[...]


_dense_in = pl.pallas_call(
    _dense_in_body,
    grid=(GRID,),
    in_specs=[
        pl.BlockSpec((BR, C), lambda i: (i, 0)),
        pl.BlockSpec((C, C), lambda i: (0, 0)),
        pl.BlockSpec((C, C), lambda i: (0, 0)),
        pl.BlockSpec((1, C), lambda i: (0, 0)),
    ],
    out_specs=_zspec + [pl.BlockSpec((BR, C), lambda i: (i, 0))],
    out_shape=_zshape + [jax.ShapeDtypeStruct((NP, C), _f32)],
)


def _combine(acc_ref, deg_ref, r_ref):
    d = jnp.maximum(deg_ref[...], 1.0)
    agg = jnp.concatenate([acc_ref[0], acc_ref[1]], axis=-1) / d[:, None]
    return agg + r_ref[...]


def _dense_mid_body(acc_ref, deg_ref, r_ref, wl_ref, wr_ref, b_ref,
                    z0_ref, z1_ref, r2_ref):
    h = jnp.maximum(_combine(acc_ref, deg_ref, r_ref), 0.0)
    z = jnp.dot(h, wl_ref[...], preferred_element_type=_f32)
    z0_ref[...] = z[:, :CHH]
    z1_ref[...] = z[:, CHH:]
    r2_ref[...] = jnp.dot(h, wr_ref[...], preferred_element_type=_f32) + b_ref[...]


_dense_mid = pl.pallas_call(
    _dense_mid_body,
    grid=(GRID,),
    in_specs=[
        pl.BlockSpec((NC, BR, CHH), lambda i: (0, i, 0)),
        pl.BlockSpec((BR,), lambda i: (i,)),
        pl.BlockSpec((BR, C), lambda i: (i, 0)),
        pl.BlockSpec((C, C), lambda i: (0, 0)),
        pl.BlockSpec((C, C), lambda i: (0, 0)),
        pl.BlockSpec((1, C), lambda i: (0, 0)),
    ],
    out_specs=_zspec + [pl.BlockSpec((BR, C), lambda i: (i, 0))],
    out_shape=_zshape + [jax.ShapeDtypeStruct((NP, C), _f32)],
)


def _dense_out_body(acc_ref, deg_ref, r_ref, o_ref):
    o = _combine(acc_ref, deg_ref, r_ref)
    m = jnp.max(o, axis=-1, keepdims=True)
    s = jnp.sum(jnp.exp(o - m), axis=-1, keepdims=True)
    o_ref[...] = (o - m) - jnp.log(s)


_dense_out = pl.pallas_call(
    _dense_out_body,
    grid=(GRID,),
    in_specs=[
        pl.BlockSpec((NC, BR, CHH), lambda i: (0, i, 0)),
        pl.BlockSpec((BR,), lambda i: (i,)),
        pl.BlockSpec((BR, C), lambda i: (i, 0)),
    ],
    out_specs=pl.BlockSpec((BR, C), lambda i: (i, 0)),
    out_shape=jax.ShapeDtypeStruct((NP, C), _f32),
)


@jax.jit
def kernel(x, edge_index, W1_l, W1_r, b1, W2_l, W2_r, b2):
    src = edge_index[0].astype(jnp.int32)
    dst = edge_index[1].astype(jnp.int32)
    pad = EP - E
    src_p = jnp.concatenate([src, jnp.zeros((pad,), jnp.int32)]).reshape(NS * CPW, 1, CH)
    dst_p = jnp.concatenate([dst, jnp.full((pad,), TRASH, jnp.int32)]).reshape(NS * CPW, 1, CH)
    ei_p = jnp.concatenate([src_p, dst_p], axis=1)
    x_p = jnp.concatenate([x, jnp.zeros((NP - N, C), _f32)], axis=0)
    zrows = jnp.zeros((ZR, CHH), _f32)
    zvec = jnp.zeros((ZR,), _f32)
    b1r = b1.reshape(1, C)
    b2r = b2.reshape(1, C)

    z1a, z1b, r1 = _dense_in(x_p, W1_l, W1_r, b1r)
    acc1, deg = _scatter_deg(z1a, z1b, ei_p, zrows, zvec)
    z2a, z2b, r2 = _dense_mid(acc1, deg, r1, W2_l, W2_r, b2r)
    (acc2,) = _scatter(z2a, z2b, ei_p, zrows, zvec)
    out = _dense_out(acc2, deg, r2)
    return out[:N]
